# baseline (speedup 1.0000x reference)
import functools

import jax
import jax.numpy as jnp
from jax import lax
from jax.experimental import pallas as pl
from jax.experimental.pallas import tpu as pltpu
from jax.experimental.pallas import tpu_sc as plsc

N_GROUPS = 10000
N_MOD = 320000
D = 128
D_PROJ = 8

NC = 2
NS = 16
NW = NC * NS
GPW = 320
NB = GPW // 16
NG_PAD = NW * GPW
CSR_PAD = NG_PAD + 16
CHUNK = 8192
NEG_INF = float("-inf")


def _floor16(x):
    return pl.multiple_of((x // 16) * 16, 16)


def _sc_body(xmod, vals, csr, outp, outs, buf, csr_v, cl_v, seen_v, rows_v,
             sem):
    cid = lax.axis_index("c")
    sid = lax.axis_index("s")
    wid = sid * NC + cid
    base_g = pl.multiple_of(wid * GPW, GPW)

    pltpu.sync_copy(csr.at[pl.ds(base_g, GPW + 16)], csr_v)

    zeros16i = jnp.zeros((16,), jnp.int32)
    zeros16f = jnp.zeros((16,), jnp.float32)
    lanes = jax.lax.iota(jnp.int32, 16)

    for j in range(320 // 16, 384 // 16):
        cl_v[pl.ds(j * 16, 16)] = zeros16i

    s0 = csr_v[pl.ds(0, 16)][0]
    w0 = _floor16(jnp.minimum(s0, N_MOD - CHUNK))
    pltpu.sync_copy(vals.at[pl.ds(w0, CHUNK)], buf)

    def block_body(b, w_blk):
        s_vec = csr_v[pl.ds(b * 16, 16)]
        e_vec = csr_v[pl.ds(b * 16 + 1, 16)]
        counts = e_vec - s_vec
        s0b = s_vec[0]
        e15 = e_vec[15]
        fits = (e15 - s0b) <= (CHUNK - 16)

        def fast_path():
            refill = e15 > w_blk + CHUNK
            w1 = pl.multiple_of(
                jnp.where(refill, _floor16(jnp.minimum(s0b, N_MOD - CHUNK)),
                          w_blk), 16)

            @pl.when(refill)
            def _():
                pltpu.sync_copy(vals.at[pl.ds(w1, CHUNK)], buf)

            nmax = jnp.max(counts)

            def step(t, carry):
                bv, bi = carry
                mask = counts > t
                idx = s_vec + t
                roff = idx - w1
                v = plsc.load_gather(buf, [roff], mask=mask)
                vm = jnp.where(mask, v, NEG_INF)
                better = vm > bv
                bv = jnp.where(better, vm, bv)
                bi = jnp.where(better, idx, bi)
                return (bv, bi)

            bv, bi = lax.fori_loop(
                0, nmax, step,
                (jnp.full((16,), NEG_INF, jnp.float32),
                 jnp.full((16,), N_MOD, jnp.int32)))
            cl_vec = jnp.where(bi >= N_MOD, 0, bi)
            seen_vec = jnp.where(counts > 0, 1, 0)
            return (w1, cl_vec, seen_vec)

        def slow_path():
            def group_body(i, carry):
                w_cur, cl_acc, seen_acc = carry
                g = b * 16 + i
                s = csr_v[pl.ds(g, 16)][0]
                e = csr_v[pl.ds(g + 1, 16)][0]
                nk = (e - s + 15) // 16

                def chunk_body(k, kcarry):
                    w, bv, bi = kcarry
                    c0 = s + k * 16
                    rf = jnp.logical_and(
                        c0 + 16 > w + CHUNK, w < N_MOD - CHUNK)
                    wn = pl.multiple_of(
                        jnp.where(rf, _floor16(jnp.minimum(c0, N_MOD - CHUNK)),
                                  w), 16)

                    @pl.when(rf)
                    def _():
                        pltpu.sync_copy(vals.at[pl.ds(wn, CHUNK)], buf)

                    il = c0 + lanes
                    mask = il < e
                    v = plsc.load_gather(buf, [il - wn], mask=mask)
                    vm = jnp.where(mask, v, NEG_INF)
                    better = vm > bv
                    bv = jnp.where(better, vm, bv)
                    bi = jnp.where(better, il, bi)
                    return (wn, bv, bi)

                w_f, bv, bi = lax.fori_loop(
                    0, nk, chunk_body,
                    (w_cur, jnp.full((16,), NEG_INF, jnp.float32),
                     jnp.full((16,), N_MOD, jnp.int32)))

                m = jnp.max(bv)
                cand = jnp.where(bv == m, bi, jnp.int32(N_MOD))
                a = jnp.min(cand)
                cl = jnp.where(a >= N_MOD, jnp.int32(0), a)
                sn = jnp.where(e > s, jnp.int32(1), jnp.int32(0))
                cl_acc = jnp.where(lanes == i, cl, cl_acc)
                seen_acc = jnp.where(lanes == i, sn, seen_acc)
                return (w_f, cl_acc, seen_acc)

            return lax.fori_loop(0, 16, group_body,
                                 (w_blk, zeros16i, zeros16i))

        w_f, cl_vec, seen_vec = lax.cond(fits, fast_path, slow_path)
        cl_v[pl.ds(b * 16, 16)] = cl_vec
        seen_v[pl.ds(b * 16, 16)] = seen_vec
        return w_f

    with jax.named_scope("p1scan"):
        lax.fori_loop(0, NB, block_body, w0)

    with jax.named_scope("p2gather"):
        copies = [
            pltpu.async_copy(
                xmod.at[cl_v.at[pl.ds(j * 128, 128)]],
                rows_v.at[pl.ds(j * 128, 128)], sem)
            for j in range(3)
        ]
        for c in copies:
            c.wait()

    def zero_blk(b, _):
        sv = seen_v[pl.ds(b * 16, 16)]
        anyz = jnp.min(sv)

        @pl.when(anyz == 0)
        def _():
            def zero_one(i, __):
                g = b * 16 + i
                sn = seen_v[pl.ds(g, 16)][0]

                @pl.when(sn == 0)
                def _():
                    for k in range(8):
                        rows_v[g, pl.ds(k * 16, 16)] = zeros16f
                return 0

            lax.fori_loop(0, 16, zero_one, 0)
        return 0

    with jax.named_scope("p3zero"):
        lax.fori_loop(0, NB, zero_blk, 0)

    with jax.named_scope("p4out"):
        pltpu.sync_copy(rows_v.at[pl.ds(0, GPW)], outp.at[pl.ds(base_g, GPW)])
        pltpu.sync_copy(seen_v.at[pl.ds(0, GPW)],
                        outs.at[pl.ds(base_g, GPW)])


@functools.partial(
    pl.kernel,
    out_type=(
        jax.ShapeDtypeStruct((NG_PAD, D), jnp.float32),
        jax.ShapeDtypeStruct((NG_PAD,), jnp.int32),
    ),
    scratch_types=[
        pltpu.VMEM((CHUNK,), jnp.float32),
        pltpu.VMEM((GPW + 16,), jnp.int32),
        pltpu.VMEM((3 * 128,), jnp.int32),
        pltpu.VMEM((GPW + 16,), jnp.int32),
        pltpu.VMEM((3 * 128, D), jnp.float32),
        pltpu.SemaphoreType.DMA,
    ],
    mesh=plsc.VectorSubcoreMesh(core_axis_name="c", subcore_axis_name="s"),
    compiler_params=pltpu.CompilerParams(
        needs_layout_passes=False, use_tc_tiling_on_sc=False),
)
def _csr_pool_sc(xmod, vals, csr, outp, outs, *scratch):
    _sc_body(xmod, vals, csr, outp, outs, *scratch)


def kernel(x_main, x_mod, x_proj, csr_idx):
    del x_main
    vals = x_proj[:, 0]
    csr_pad = jnp.concatenate(
        [csr_idx,
         jnp.full((CSR_PAD - (N_GROUPS + 1),), N_MOD, dtype=jnp.int32)])
    pool_pad, seen_pad = _csr_pool_sc(x_mod, vals, csr_pad)
    return pool_pad[:N_GROUPS], seen_pad[:N_GROUPS] != 0


# speedup vs baseline: 2.1356x; 2.1356x over previous
import functools

import jax
import jax.numpy as jnp
from jax import lax
from jax.experimental import pallas as pl
from jax.experimental.pallas import tpu as pltpu
from jax.experimental.pallas import tpu_sc as plsc

N_GROUPS = 10000
N_MOD = 320000
D = 128
D_PROJ = 8

NC = 2
NS = 16
NW = NC * NS
GPW = 320
NB = GPW // 16
NG_PAD = NW * GPW
CSR_PAD = NG_PAD + 16
CHUNK = 8192
NEG_INF = float("-inf")


def _floor16(x):
    return pl.multiple_of((x // 16) * 16, 16)


def _sc_body(xmod, vals, csr, outp, outs, buf, csr_v, cl_v, seen_v, rows_v,
             sem):
    cid = lax.axis_index("c")
    sid = lax.axis_index("s")
    wid = sid * NC + cid
    base_g = pl.multiple_of(wid * GPW, GPW)

    pltpu.sync_copy(csr.at[pl.ds(base_g, GPW + 16)], csr_v)

    zeros16i = jnp.zeros((16,), jnp.int32)
    zeros16f = jnp.zeros((16,), jnp.float32)
    lanes = jax.lax.iota(jnp.int32, 16)

    s0 = csr_v[pl.ds(0, 16)][0]
    w0 = _floor16(jnp.minimum(s0, N_MOD - CHUNK))
    pltpu.sync_copy(vals.at[pl.ds(w0, CHUNK)], buf)

    def block_body(b, w_blk):
        s_vec = csr_v[pl.ds(b * 16, 16)]
        e_vec = csr_v[pl.ds(b * 16 + 1, 16)]
        counts = e_vec - s_vec
        s0b = s_vec[0]
        e15 = e_vec[15]
        fits = (e15 - s0b) <= (CHUNK - 16)

        def fast_path():
            refill = e15 > w_blk + CHUNK
            w1 = pl.multiple_of(
                jnp.where(refill, _floor16(jnp.minimum(s0b, N_MOD - CHUNK)),
                          w_blk), 16)

            @pl.when(refill)
            def _():
                pltpu.sync_copy(vals.at[pl.ds(w1, CHUNK)], buf)

            nmax = jnp.max(counts)

            def step(t, carry):
                bv, bi = carry
                mask = counts > t
                idx = s_vec + t
                roff = idx - w1
                v = plsc.load_gather(buf, [roff], mask=mask)
                vm = jnp.where(mask, v, NEG_INF)
                better = vm > bv
                bv = jnp.where(better, vm, bv)
                bi = jnp.where(better, idx, bi)
                return (bv, bi)

            bv, bi = lax.fori_loop(
                0, nmax, step,
                (jnp.full((16,), NEG_INF, jnp.float32),
                 jnp.full((16,), N_MOD, jnp.int32)))
            cl_vec = jnp.where(bi >= N_MOD, 0, bi)
            seen_vec = jnp.where(counts > 0, 1, 0)
            return (w1, cl_vec, seen_vec)

        def slow_path():
            def group_body(i, carry):
                w_cur, cl_acc, seen_acc = carry
                g = b * 16 + i
                s = csr_v[pl.ds(g, 16)][0]
                e = csr_v[pl.ds(g + 1, 16)][0]
                nk = (e - s + 15) // 16

                def chunk_body(k, kcarry):
                    w, bv, bi = kcarry
                    c0 = s + k * 16
                    rf = jnp.logical_and(
                        c0 + 16 > w + CHUNK, w < N_MOD - CHUNK)
                    wn = pl.multiple_of(
                        jnp.where(rf, _floor16(jnp.minimum(c0, N_MOD - CHUNK)),
                                  w), 16)

                    @pl.when(rf)
                    def _():
                        pltpu.sync_copy(vals.at[pl.ds(wn, CHUNK)], buf)

                    il = c0 + lanes
                    mask = il < e
                    v = plsc.load_gather(buf, [il - wn], mask=mask)
                    vm = jnp.where(mask, v, NEG_INF)
                    better = vm > bv
                    bv = jnp.where(better, vm, bv)
                    bi = jnp.where(better, il, bi)
                    return (wn, bv, bi)

                w_f, bv, bi = lax.fori_loop(
                    0, nk, chunk_body,
                    (w_cur, jnp.full((16,), NEG_INF, jnp.float32),
                     jnp.full((16,), N_MOD, jnp.int32)))

                m = jnp.max(bv)
                cand = jnp.where(bv == m, bi, jnp.int32(N_MOD))
                a = jnp.min(cand)
                cl = jnp.where(a >= N_MOD, jnp.int32(0), a)
                sn = jnp.where(e > s, jnp.int32(1), jnp.int32(0))
                cl_acc = jnp.where(lanes == i, cl, cl_acc)
                seen_acc = jnp.where(lanes == i, sn, seen_acc)
                return (w_f, cl_acc, seen_acc)

            return lax.fori_loop(0, 16, group_body,
                                 (w_blk, zeros16i, zeros16i))

        w_f, cl_vec, seen_vec = lax.cond(fits, fast_path, slow_path)
        cl_v[pl.ds(b * 16, 16)] = cl_vec
        seen_v[pl.ds(b * 16, 16)] = seen_vec
        for j in range(16):
            pltpu.async_copy(
                xmod.at[pl.ds(cl_vec[j], 1)],
                rows_v.at[pl.ds(b * 16 + j, 1)], sem)
        return w_f

    with jax.named_scope("p1scan"):
        lax.fori_loop(0, NB, block_body, w0)

    with jax.named_scope("p2gather"):
        def drain(i, _):
            pltpu.make_async_copy(
                xmod.at[pl.ds(0, 1)], rows_v.at[pl.ds(0, 1)], sem).wait()
            return 0

        lax.fori_loop(0, GPW, drain, 0)

    def zero_blk(b, _):
        sv = seen_v[pl.ds(b * 16, 16)]
        anyz = jnp.min(sv)

        @pl.when(anyz == 0)
        def _():
            def zero_one(i, __):
                g = b * 16 + i
                sn = seen_v[pl.ds(g, 16)][0]

                @pl.when(sn == 0)
                def _():
                    for k in range(8):
                        rows_v[g, pl.ds(k * 16, 16)] = zeros16f
                return 0

            lax.fori_loop(0, 16, zero_one, 0)
        return 0

    with jax.named_scope("p3zero"):
        lax.fori_loop(0, NB, zero_blk, 0)

    with jax.named_scope("p4out"):
        pltpu.sync_copy(rows_v.at[pl.ds(0, GPW)], outp.at[pl.ds(base_g, GPW)])
        pltpu.sync_copy(seen_v.at[pl.ds(0, GPW)],
                        outs.at[pl.ds(base_g, GPW)])


@functools.partial(
    pl.kernel,
    out_type=(
        jax.ShapeDtypeStruct((NG_PAD, D), jnp.float32),
        jax.ShapeDtypeStruct((NG_PAD,), jnp.int32),
    ),
    scratch_types=[
        pltpu.VMEM((CHUNK,), jnp.float32),
        pltpu.VMEM((GPW + 16,), jnp.int32),
        pltpu.VMEM((GPW,), jnp.int32),
        pltpu.VMEM((GPW + 16,), jnp.int32),
        pltpu.VMEM((GPW, D), jnp.float32),
        pltpu.SemaphoreType.DMA,
    ],
    mesh=plsc.VectorSubcoreMesh(core_axis_name="c", subcore_axis_name="s"),
    compiler_params=pltpu.CompilerParams(
        needs_layout_passes=False, use_tc_tiling_on_sc=False),
)
def _csr_pool_sc(xmod, vals, csr, outp, outs, *scratch):
    _sc_body(xmod, vals, csr, outp, outs, *scratch)


def kernel(x_main, x_mod, x_proj, csr_idx):
    del x_main
    vals = x_proj[:, 0]
    csr_pad = jnp.concatenate(
        [csr_idx,
         jnp.full((CSR_PAD - (N_GROUPS + 1),), N_MOD, dtype=jnp.int32)])
    pool_pad, seen_pad = _csr_pool_sc(x_mod, vals, csr_pad)
    return pool_pad[:N_GROUPS], seen_pad[:N_GROUPS] != 0


# speedup vs baseline: 2.6559x; 1.2436x over previous
import functools

import jax
import jax.numpy as jnp
from jax import lax
from jax.experimental import pallas as pl
from jax.experimental.pallas import tpu as pltpu
from jax.experimental.pallas import tpu_sc as plsc

N_GROUPS = 10000
N_MOD = 320000
D = 128
D_PROJ = 8

NC = 2
NS = 16
NW = NC * NS
GPW = 320
NB = GPW // 16
NG_PAD = NW * GPW
CSR_PAD = NG_PAD + 16
CHUNK = 8192
NEG_INF = float("-inf")


def _floor128(x):
    return pl.multiple_of((x // 128) * 128, 128)


def _sc_body(xmod, vals, csr, outp, outs, buf, csr_v, cl_v, seen_v, rows_v,
             sem):
    cid = lax.axis_index("c")
    sid = lax.axis_index("s")
    wid = sid * NC + cid
    base_g = pl.multiple_of(wid * GPW, GPW)

    pltpu.sync_copy(csr.at[pl.ds(base_g, GPW + 16)], csr_v)

    zeros16i = jnp.zeros((16,), jnp.int32)
    zeros16f = jnp.zeros((16,), jnp.float32)
    lanes = jax.lax.iota(jnp.int32, 16)

    def fill_window(w):
        pltpu.sync_copy(
            vals.at[pl.ds(w // 128, CHUNK // 128), pl.ds(0, 1), :], buf)

    s0 = csr_v[pl.ds(0, 16)][0]
    w0 = _floor128(jnp.minimum(s0, N_MOD - CHUNK))
    fill_window(w0)

    def block_body(b, w_blk):
        s_vec = csr_v[pl.ds(b * 16, 16)]
        e_vec = csr_v[pl.ds(b * 16 + 1, 16)]
        counts = e_vec - s_vec
        s0b = s_vec[0]
        e15 = e_vec[15]
        fits = (e15 - s0b) <= (CHUNK - 128)

        def fast_path():
            refill = e15 > w_blk + CHUNK
            w1 = pl.multiple_of(
                jnp.where(refill, _floor128(jnp.minimum(s0b, N_MOD - CHUNK)),
                          w_blk), 128)

            @pl.when(refill)
            def _():
                fill_window(w1)

            nmax = jnp.max(counts)

            def step(t, carry):
                bv, bi = carry
                mask = counts > t
                idx = s_vec + t
                roff = idx - w1
                v = plsc.load_gather(
                    buf, [roff >> 7, zeros16i, roff & 127], mask=mask)
                vm = jnp.where(mask, v, NEG_INF)
                better = vm > bv
                bv = jnp.where(better, vm, bv)
                bi = jnp.where(better, idx, bi)
                return (bv, bi)

            bv, bi = lax.fori_loop(
                0, nmax, step,
                (jnp.full((16,), NEG_INF, jnp.float32),
                 jnp.full((16,), N_MOD, jnp.int32)))
            cl_vec = jnp.where(bi >= N_MOD, 0, bi)
            seen_vec = jnp.where(counts > 0, 1, 0)
            return (w1, cl_vec, seen_vec)

        def slow_path():
            def group_body(i, carry):
                w_cur, cl_acc, seen_acc = carry
                g = b * 16 + i
                s = csr_v[pl.ds(g, 16)][0]
                e = csr_v[pl.ds(g + 1, 16)][0]
                nk = (e - s + 15) // 16

                def chunk_body(k, kcarry):
                    w, bv, bi = kcarry
                    c0 = s + k * 16
                    rf = jnp.logical_and(
                        c0 + 16 > w + CHUNK, w < N_MOD - CHUNK)
                    wn = pl.multiple_of(
                        jnp.where(
                            rf, _floor128(jnp.minimum(c0, N_MOD - CHUNK)),
                            w), 128)

                    @pl.when(rf)
                    def _():
                        fill_window(wn)

                    il = c0 + lanes
                    mask = il < e
                    roff = il - wn
                    v = plsc.load_gather(
                        buf, [roff >> 7, zeros16i, roff & 127], mask=mask)
                    vm = jnp.where(mask, v, NEG_INF)
                    better = vm > bv
                    bv = jnp.where(better, vm, bv)
                    bi = jnp.where(better, il, bi)
                    return (wn, bv, bi)

                w_f, bv, bi = lax.fori_loop(
                    0, nk, chunk_body,
                    (w_cur, jnp.full((16,), NEG_INF, jnp.float32),
                     jnp.full((16,), N_MOD, jnp.int32)))

                m = jnp.max(bv)
                cand = jnp.where(bv == m, bi, jnp.int32(N_MOD))
                a = jnp.min(cand)
                cl = jnp.where(a >= N_MOD, jnp.int32(0), a)
                sn = jnp.where(e > s, jnp.int32(1), jnp.int32(0))
                cl_acc = jnp.where(lanes == i, cl, cl_acc)
                seen_acc = jnp.where(lanes == i, sn, seen_acc)
                return (w_f, cl_acc, seen_acc)

            return lax.fori_loop(0, 16, group_body,
                                 (w_blk, zeros16i, zeros16i))

        w_f, cl_vec, seen_vec = lax.cond(fits, fast_path, slow_path)
        cl_v[pl.ds(b * 16, 16)] = cl_vec
        seen_v[pl.ds(b * 16, 16)] = seen_vec
        for j in range(16):
            pltpu.async_copy(
                xmod.at[pl.ds(cl_vec[j], 1)],
                rows_v.at[pl.ds(b * 16 + j, 1)], sem)
        return w_f

    with jax.named_scope("p1scan"):
        lax.fori_loop(0, NB, block_body, w0)

    with jax.named_scope("p2gather"):
        def drain(i, _):
            pltpu.make_async_copy(
                xmod.at[pl.ds(0, 1)], rows_v.at[pl.ds(0, 1)], sem).wait()
            return 0

        lax.fori_loop(0, GPW, drain, 0)

    def zero_blk(b, _):
        sv = seen_v[pl.ds(b * 16, 16)]
        anyz = jnp.min(sv)

        @pl.when(anyz == 0)
        def _():
            def zero_one(i, __):
                g = b * 16 + i
                sn = seen_v[pl.ds(g, 16)][0]

                @pl.when(sn == 0)
                def _():
                    for k in range(8):
                        rows_v[g, pl.ds(k * 16, 16)] = zeros16f
                return 0

            lax.fori_loop(0, 16, zero_one, 0)
        return 0

    with jax.named_scope("p3zero"):
        lax.fori_loop(0, NB, zero_blk, 0)

    with jax.named_scope("p4out"):
        pltpu.sync_copy(rows_v.at[pl.ds(0, GPW)], outp.at[pl.ds(base_g, GPW)])
        pltpu.sync_copy(seen_v.at[pl.ds(0, GPW)],
                        outs.at[pl.ds(base_g, GPW)])


@functools.partial(
    pl.kernel,
    out_type=(
        jax.ShapeDtypeStruct((NG_PAD, D), jnp.float32),
        jax.ShapeDtypeStruct((NG_PAD,), jnp.int32),
    ),
    scratch_types=[
        pltpu.VMEM((CHUNK // 128, 1, 128), jnp.float32),
        pltpu.VMEM((GPW + 16,), jnp.int32),
        pltpu.VMEM((GPW,), jnp.int32),
        pltpu.VMEM((GPW + 16,), jnp.int32),
        pltpu.VMEM((GPW, D), jnp.float32),
        pltpu.SemaphoreType.DMA,
    ],
    mesh=plsc.VectorSubcoreMesh(core_axis_name="c", subcore_axis_name="s"),
    compiler_params=pltpu.CompilerParams(
        needs_layout_passes=False, use_tc_tiling_on_sc=False),
)
def _csr_pool_sc(xmod, vals, csr, outp, outs, *scratch):
    _sc_body(xmod, vals, csr, outp, outs, *scratch)


def kernel(x_main, x_mod, x_proj, csr_idx):
    del x_main
    vals3 = jnp.transpose(x_proj.reshape(N_MOD // 128, 128, D_PROJ),
                          (0, 2, 1))
    csr_pad = jnp.concatenate(
        [csr_idx,
         jnp.full((CSR_PAD - (N_GROUPS + 1),), N_MOD, dtype=jnp.int32)])
    pool_pad, seen_pad = _csr_pool_sc(x_mod, vals3, csr_pad)
    return pool_pad[:N_GROUPS], seen_pad[:N_GROUPS] != 0


# speedup vs baseline: 2.8914x; 1.0887x over previous
import functools

import jax
import jax.numpy as jnp
from jax import lax
from jax.experimental import pallas as pl
from jax.experimental.pallas import tpu as pltpu
from jax.experimental.pallas import tpu_sc as plsc

N_GROUPS = 10000
N_MOD = 320000
D = 128
D_PROJ = 8

NC = 2
NS = 16
NW = NC * NS
GPW = 320
NB = GPW // 16
NG_PAD = NW * GPW
CSR_PAD = NG_PAD + 16
CHUNK = 8192
NEG_INF = float("-inf")


def _floor128(x):
    return pl.multiple_of((x // 128) * 128, 128)


def _sc_body(xmod, vals, csr, outp, outs, buf, csr_v, cl_v, seen_v, rows_v,
             sem):
    cid = lax.axis_index("c")
    sid = lax.axis_index("s")
    wid = sid * NC + cid
    base_g = pl.multiple_of(wid * GPW, GPW)

    pltpu.sync_copy(csr.at[pl.ds(base_g, GPW + 16)], csr_v)

    zeros16i = jnp.zeros((16,), jnp.int32)
    zeros16f = jnp.zeros((16,), jnp.float32)
    lanes = jax.lax.iota(jnp.int32, 16)

    def fill_window(w):
        pltpu.sync_copy(
            vals.at[pl.ds(w // 128, CHUNK // 128), pl.ds(0, 1), :], buf)

    s0 = csr_v[pl.ds(0, 16)][0]
    w0 = _floor128(jnp.minimum(s0, N_MOD - CHUNK))
    fill_window(w0)

    def block_body(b, w_blk):
        s_vec = csr_v[pl.ds(b * 16, 16)]
        e_vec = csr_v[pl.ds(b * 16 + 1, 16)]
        counts = e_vec - s_vec
        s0b = s_vec[0]
        e15 = e_vec[15]
        fits = (e15 - s0b) <= (CHUNK - 128)

        def fast_path():
            refill = e15 > w_blk + CHUNK
            w1 = pl.multiple_of(
                jnp.where(refill, _floor128(jnp.minimum(s0b, N_MOD - CHUNK)),
                          w_blk), 128)

            @pl.when(refill)
            def _():
                fill_window(w1)

            nmax = jnp.max(counts)

            def step(t, carry):
                bv, bi = carry
                mask = counts > t
                idx = s_vec + t
                roff = idx - w1
                v = plsc.load_gather(
                    buf, [roff >> 7, zeros16i, roff & 127], mask=mask)
                vm = jnp.where(mask, v, NEG_INF)
                better = vm > bv
                bv = jnp.where(better, vm, bv)
                bi = jnp.where(better, idx, bi)
                return (bv, bi)

            bv, bi = lax.fori_loop(
                0, nmax, step,
                (jnp.full((16,), NEG_INF, jnp.float32),
                 jnp.full((16,), N_MOD, jnp.int32)))
            cl_vec = jnp.where(bi >= N_MOD, 0, bi)
            seen_vec = jnp.where(counts > 0, 1, 0)
            return (w1, cl_vec, seen_vec)

        def slow_path():
            def group_body(i, carry):
                w_cur, cl_acc, seen_acc = carry
                g = b * 16 + i
                s = csr_v[pl.ds(g, 16)][0]
                e = csr_v[pl.ds(g + 1, 16)][0]
                nk = (e - s + 15) // 16

                def chunk_body(k, kcarry):
                    w, bv, bi = kcarry
                    c0 = s + k * 16
                    rf = jnp.logical_and(
                        c0 + 16 > w + CHUNK, w < N_MOD - CHUNK)
                    wn = pl.multiple_of(
                        jnp.where(
                            rf, _floor128(jnp.minimum(c0, N_MOD - CHUNK)),
                            w), 128)

                    @pl.when(rf)
                    def _():
                        fill_window(wn)

                    il = c0 + lanes
                    mask = il < e
                    roff = il - wn
                    v = plsc.load_gather(
                        buf, [roff >> 7, zeros16i, roff & 127], mask=mask)
                    vm = jnp.where(mask, v, NEG_INF)
                    better = vm > bv
                    bv = jnp.where(better, vm, bv)
                    bi = jnp.where(better, il, bi)
                    return (wn, bv, bi)

                w_f, bv, bi = lax.fori_loop(
                    0, nk, chunk_body,
                    (w_cur, jnp.full((16,), NEG_INF, jnp.float32),
                     jnp.full((16,), N_MOD, jnp.int32)))

                m = jnp.max(bv)
                cand = jnp.where(bv == m, bi, jnp.int32(N_MOD))
                a = jnp.min(cand)
                cl = jnp.where(a >= N_MOD, jnp.int32(0), a)
                sn = jnp.where(e > s, jnp.int32(1), jnp.int32(0))
                cl_acc = jnp.where(lanes == i, cl, cl_acc)
                seen_acc = jnp.where(lanes == i, sn, seen_acc)
                return (w_f, cl_acc, seen_acc)

            return lax.fori_loop(0, 16, group_body,
                                 (w_blk, zeros16i, zeros16i))

        w_f, cl_vec, seen_vec = lax.cond(fits, fast_path, slow_path)
        cl_v[pl.ds(b * 16, 16)] = cl_vec
        seen_v[pl.ds(b * 16, 16)] = seen_vec
        for j in range(16):
            pltpu.async_copy(
                xmod.at[pl.ds(cl_vec[j], 1)],
                rows_v.at[pl.ds(b * 16 + j, 1)], sem)
        return w_f

    with jax.named_scope("p1scan"):
        lax.fori_loop(0, NB, block_body, w0)

    with jax.named_scope("p2gather"):
        def drain(i, _):
            pltpu.make_async_copy(
                xmod.at[pl.ds(0, 1)], rows_v.at[pl.ds(0, 1)], sem).wait()
            return 0

        lax.fori_loop(0, GPW, drain, 0)

    def zero_blk(b, _):
        sv = seen_v[pl.ds(b * 16, 16)]
        anyz = jnp.min(sv)

        @pl.when(anyz == 0)
        def _():
            def zero_one(i, __):
                g = b * 16 + i
                sn = seen_v[pl.ds(g, 16)][0]

                @pl.when(sn == 0)
                def _():
                    for k in range(8):
                        rows_v[g, pl.ds(k * 16, 16)] = zeros16f
                return 0

            lax.fori_loop(0, 16, zero_one, 0)
        return 0

    with jax.named_scope("p3zero"):
        lax.fori_loop(0, NB, zero_blk, 0)

    with jax.named_scope("p4out"):
        @pl.when(wid < NW - 1)
        def _():
            pltpu.sync_copy(rows_v.at[pl.ds(0, GPW)],
                            outp.at[pl.ds(base_g, GPW)])
            pltpu.sync_copy(seen_v.at[pl.ds(0, GPW)],
                            outs.at[pl.ds(base_g, GPW)])

        @pl.when(wid == NW - 1)
        def _():
            tail = N_GROUPS - (NW - 1) * GPW
            pltpu.sync_copy(rows_v.at[pl.ds(0, tail)],
                            outp.at[pl.ds(base_g, tail)])
            pltpu.sync_copy(seen_v.at[pl.ds(0, tail)],
                            outs.at[pl.ds(base_g, tail)])


@functools.partial(
    pl.kernel,
    out_type=(
        jax.ShapeDtypeStruct((N_GROUPS, D), jnp.float32),
        jax.ShapeDtypeStruct((N_GROUPS,), jnp.int32),
    ),
    scratch_types=[
        pltpu.VMEM((CHUNK // 128, 1, 128), jnp.float32),
        pltpu.VMEM((GPW + 16,), jnp.int32),
        pltpu.VMEM((GPW,), jnp.int32),
        pltpu.VMEM((GPW + 16,), jnp.int32),
        pltpu.VMEM((GPW, D), jnp.float32),
        pltpu.SemaphoreType.DMA,
    ],
    mesh=plsc.VectorSubcoreMesh(core_axis_name="c", subcore_axis_name="s"),
    compiler_params=pltpu.CompilerParams(
        needs_layout_passes=False, use_tc_tiling_on_sc=False),
)
def _csr_pool_sc(xmod, vals, csr, outp, outs, *scratch):
    _sc_body(xmod, vals, csr, outp, outs, *scratch)


def kernel(x_main, x_mod, x_proj, csr_idx):
    del x_main
    vals3 = jnp.transpose(x_proj.reshape(N_MOD // 128, 128, D_PROJ),
                          (0, 2, 1))
    csr_pad = jnp.concatenate(
        [csr_idx,
         jnp.full((CSR_PAD - (N_GROUPS + 1),), N_MOD, dtype=jnp.int32)])
    pool, seen = _csr_pool_sc(x_mod, vals3, csr_pad)
    return pool, seen != 0
